# baseline (speedup 1.0000x reference)
import functools

import jax
import jax.numpy as jnp
import numpy as np
from jax import lax
from jax.experimental import pallas as pl
from jax.experimental.pallas import tpu as pltpu
from jax.experimental.pallas import tpu_sc as plsc

_NW = 32

_PREC = jax.lax.Precision.DEFAULT
EPS = 1e-15
N_RBF = 6
CUTOFF = 5.0
P_ENV = 6
L_SPHER = 7
N_SPHER = 6
D = 128
N_BIL = 8
N_CONV = 2
N_SBF = L_SPHER * N_SPHER

_BLK = 1000


def _swish(x):
    return x * jax.nn.sigmoid(x)


def _rowblock_call(fn, row_args, full_args, out_dims, block=_BLK):
    full_args = [a[None] if a.ndim == 1 else a for a in full_args]
    n = row_args[0].shape[0]
    grid = n // block

    def body(*refs):
        n_in = len(row_args) + len(full_args)
        in_refs, out_refs = refs[:n_in], refs[n_in:]
        vals = [r[...] for r in in_refs]
        outs = fn(*vals)
        if not isinstance(outs, (tuple, list)):
            outs = (outs,)
        for o_ref, o in zip(out_refs, outs):
            o_ref[...] = o

    in_specs = []
    for a in row_args:
        bs = (block,) + a.shape[1:]
        in_specs.append(pl.BlockSpec(bs, lambda i, _nd=a.ndim: (i,) + (0,) * (_nd - 1)))
    for a in full_args:
        in_specs.append(pl.BlockSpec(a.shape, lambda i, _nd=a.ndim: (0,) * _nd))
    out_specs = [pl.BlockSpec((block, dm), lambda i: (i, 0)) for dm in out_dims]
    out_shape = [jax.ShapeDtypeStruct((n, dm), jnp.float32) for dm in out_dims]
    res = pl.pallas_call(
        body,
        grid=(grid,),
        in_specs=in_specs,
        out_specs=out_specs,
        out_shape=out_shape,
    )(*row_args, *full_args)
    return res


def _sc_gather(table, idx):
    t_rows, dp = table.shape
    b = idx.shape[0]
    bpw = b // _NW
    n_full, rem = divmod(bpw, 128)
    nch = n_full + (1 if rem else 0)
    mesh = plsc.VectorSubcoreMesh(core_axis_name="c", subcore_axis_name="s")

    @functools.partial(
        pl.kernel, mesh=mesh,
        out_type=jax.ShapeDtypeStruct((b, dp), jnp.float32),
        scratch_types=[
            pltpu.VMEM((bpw,), jnp.int32),
            pltpu.VMEM((128, dp), jnp.float32),
            pltpu.VMEM((128, dp), jnp.float32),
            pltpu.SemaphoreType.DMA,
            pltpu.SemaphoreType.DMA,
        ],
    )
    def k(table_h, idx_h, out_h, idx_v, buf0, buf1, sem0, sem1):
        wid = lax.axis_index("s") * 2 + lax.axis_index("c")
        base = wid * bpw
        pltpu.sync_copy(idx_h.at[pl.ds(base, bpw)], idx_v)
        bufs = (buf0, buf1)
        sems = (sem0, sem1)

        def chunk_len(j):
            return 128 if j < n_full else rem

        prev = None
        for j in range(nch):
            cl = chunk_len(j)
            cp = pltpu.async_copy(
                table_h.at[idx_v.at[pl.ds(j * 128, cl)]],
                bufs[j % 2].at[pl.ds(0, cl)], sems[j % 2])
            if prev is not None:
                prev.wait()
                pl_ = chunk_len(j - 1)
                pltpu.sync_copy(bufs[(j - 1) % 2].at[pl.ds(0, pl_)],
                                out_h.at[pl.ds(base + (j - 1) * 128, pl_)])
            prev = cp
        prev.wait()
        pl_ = chunk_len(nch - 1)
        pltpu.sync_copy(bufs[(nch - 1) % 2].at[pl.ds(0, pl_)],
                        out_h.at[pl.ds(base + (nch - 1) * 128, pl_)])

    return k(table, idx.astype(jnp.int32))


def _sc_force_scatter(fe, fa_ji, fa_jk, nbr0, nbr1, a0, a1, n_nodes):
    e = nbr0.shape[0]
    a = a0.shape[0]
    epw = e // _NW
    apw = a // _NW
    mesh = plsc.VectorSubcoreMesh(core_axis_name="c", subcore_axis_name="s")

    @functools.partial(
        pl.kernel, mesh=mesh,
        out_type=jax.ShapeDtypeStruct((_NW * n_nodes * 4,), jnp.float32),
        scratch_types=[
            pltpu.VMEM((n_nodes * 4,), jnp.float32),
            pltpu.VMEM((epw * 4,), jnp.float32),
            pltpu.VMEM((epw,), jnp.int32),
            pltpu.VMEM((epw,), jnp.int32),
            pltpu.VMEM((apw * 4,), jnp.float32),
            pltpu.VMEM((apw * 4,), jnp.float32),
            pltpu.VMEM((apw,), jnp.int32),
            pltpu.VMEM((apw,), jnp.int32),
        ],
    )
    def k(fe_h, fji_h, fjk_h, n0_h, n1_h, a0_h, a1_h, out_h,
          acc, fe_v, n0_v, n1_v, fji_v, fjk_v, a0_v, a1_v):
        wid = lax.axis_index("s") * 2 + lax.axis_index("c")
        pltpu.sync_copy(fe_h.at[pl.ds(wid * epw * 4, epw * 4)], fe_v)
        pltpu.sync_copy(n0_h.at[pl.ds(wid * epw, epw)], n0_v)
        pltpu.sync_copy(n1_h.at[pl.ds(wid * epw, epw)], n1_v)
        pltpu.sync_copy(fji_h.at[pl.ds(wid * apw * 4, apw * 4)], fji_v)
        pltpu.sync_copy(fjk_h.at[pl.ds(wid * apw * 4, apw * 4)], fjk_v)
        pltpu.sync_copy(a0_h.at[pl.ds(wid * apw, apw)], a0_v)
        pltpu.sync_copy(a1_h.at[pl.ds(wid * apw, apw)], a1_v)

        zeros16 = jnp.zeros((16,), jnp.float32)

        def zero_body(i, _):
            acc[pl.ds(i * 16, 16)] = zeros16
            return 0

        lax.fori_loop(0, n_nodes * 4 // 16, zero_body, 0)

        lane = jnp.arange(16, dtype=jnp.int32)
        row_of = lane // 4
        comp = lane % 4

        def edge_body(g, _):
            vals = fe_v[pl.ds(g * 16, 16)]
            r = g * 4 + row_of
            n0 = plsc.load_gather(n0_v, [r])
            plsc.addupdate_scatter(acc, [n0 * 4 + comp], vals)
            n1 = plsc.load_gather(n1_v, [r])
            plsc.addupdate_scatter(acc, [n1 * 4 + comp], -vals)
            return 0

        lax.fori_loop(0, epw // 4, edge_body, 0)

        def ang_body(g, _):
            vji = fji_v[pl.ds(g * 16, 16)]
            vjk = fjk_v[pl.ds(g * 16, 16)]
            r = g * 4 + row_of
            n1 = plsc.load_gather(a1_v, [r])
            t1 = n1 * 4 + comp
            plsc.addupdate_scatter(acc, [t1], vji)
            plsc.addupdate_scatter(acc, [t1], vjk)
            n0 = plsc.load_gather(a0_v, [r])
            t0 = n0 * 4 + comp
            plsc.addupdate_scatter(acc, [t0], -vji)
            plsc.addupdate_scatter(acc, [t0], -vjk)
            return 0

        lax.fori_loop(0, apw // 4, ang_body, 0)
        pltpu.sync_copy(acc, out_h.at[pl.ds(wid * n_nodes * 4, n_nodes * 4)])

    return k(fe, fa_ji, fa_jk, nbr0.astype(jnp.int32), nbr1.astype(jnp.int32),
             a0.astype(jnp.int32), a1.astype(jnp.int32))


def _partial_sum_fn(parts):
    return jnp.sum(parts, axis=0)


def _envelope(x):
    p = P_ENV
    return (1.0 - ((p + 1) * (p + 2) / 2.0) * x ** p
            + p * (p + 2) * x ** (p + 1)
            - (p * (p + 1) / 2.0) * x ** (p + 2))


def _edge_geom_fn(gi, gj):
    dx = gi[:, 0:1] - gj[:, 0:1]
    dy = gi[:, 1:2] - gj[:, 1:2]
    dz = gi[:, 2:3] - gj[:, 2:3]
    s = dx * dx + dy * dy + dz * dz
    d = jnp.sqrt(s)
    dis = jnp.sqrt(s + EPS)
    x = d / CUTOFF
    env = _envelope(x)
    c = np.sqrt(2.0 / CUTOFF)
    inv = 1.0 / (d + 1e-9)
    rbf = [c * env * jnp.sin(float(n) * np.pi * x) * inv for n in range(1, N_RBF + 1)]
    adj = [dx / dis, dy / dis, dz / dis]
    zero = jnp.zeros_like(d)
    return jnp.concatenate([d] + rbf + adj + [zero] * 6, axis=1)


def _sph_j(l, x):
    x = jnp.where(jnp.abs(x) < 1e-6, 1e-6, x)
    sx, cx = jnp.sin(x), jnp.cos(x)
    j0 = sx / x
    if l == 0:
        return j0
    j1 = sx / (x * x) - cx / x
    jm, jc = j0, j1
    for ll in range(1, l):
        jn = (2 * ll + 1) / x * jc - jm
        jm, jc = jc, jn
    return jc


def _angle_geom_fn(gi, gj, gk):
    jx = [gi[:, c:c + 1] - gj[:, c:c + 1] for c in range(3)]
    kx = [gk[:, c:c + 1] - gj[:, c:c + 1] for c in range(3)]
    s_ji = jx[0] ** 2 + jx[1] ** 2 + jx[2] ** 2
    s_jk = kx[0] ** 2 + kx[1] ** 2 + kx[2] ** 2
    d_ji = jnp.sqrt(s_ji + EPS)
    d_jk = jnp.sqrt(s_jk + EPS)
    u = [jx[c] / d_ji for c in range(3)]
    v = [kx[c] / d_jk for c in range(3)]
    cos_raw = u[0] * v[0] + u[1] * v[1] + u[2] * v[2]
    aa_ji = [(u[c] * cos_raw - v[c]) / d_ji for c in range(3)]
    aa_jk = [(v[c] * cos_raw - u[c]) / d_jk for c in range(3)]
    zero = jnp.zeros_like(d_ji)
    return jnp.concatenate(aa_ji + aa_jk + [zero] * 2, axis=1)


def _host_sbf(xyz, d, angle_list, kj_idx):
    r_ji = xyz[angle_list[:, 0]] - xyz[angle_list[:, 1]]
    r_jk = xyz[angle_list[:, 2]] - xyz[angle_list[:, 1]]
    tn_ji = ((r_ji ** 2 + EPS).sum(-1)) ** 0.5
    tn_jk = ((r_jk ** 2 + EPS).sum(-1)) ** 0.5
    cos_a = (r_ji * r_jk).sum(-1) / (tn_ji * tn_jk)
    alpha = jnp.arccos(jnp.clip(cos_a, -1.0 + 1e-7, 1.0 - 1e-7))
    x = (d[kj_idx] / CUTOFF)[:, 0]
    env = _envelope(x)
    cos_al = jnp.cos(alpha)
    P = [jnp.ones_like(cos_al), cos_al]
    for l in range(1, L_SPHER - 1):
        P.append(((2 * l + 1) * cos_al * P[l] - l * P[l - 1]) / (l + 1))
    feats = []
    for l in range(L_SPHER):
        for n in range(1, N_SPHER + 1):
            z = np.pi * (n + l / 2.0)
            feats.append(env * _sph_j(l, z * x) * P[l])
    return jnp.stack(feats, axis=-1)


def _embed_fn(ei, ej, geom, w_rbf, w_emb, b_emb):
    e_d = jnp.dot(geom[:, 1:1 + N_RBF], w_rbf, preferred_element_type=jnp.float32, precision=_PREC)
    cat = jnp.concatenate([ei, ej, e_d], axis=1)
    return _swish(jnp.dot(cat, w_emb, preferred_element_type=jnp.float32, precision=_PREC) + b_emb)


def _readout_edge_fn(m, geom, w_rbf, w0, b0, w1, b1, wh, bh, wo, bo):
    e = jnp.dot(geom[:, 1:1 + N_RBF], w_rbf, preferred_element_type=jnp.float32, precision=_PREC) * m
    e = _swish(jnp.dot(e, w0, preferred_element_type=jnp.float32, precision=_PREC) + b0)
    e = _swish(jnp.dot(e, w1, preferred_element_type=jnp.float32, precision=_PREC) + b1)
    e = _swish(jnp.dot(e, wh, preferred_element_type=jnp.float32, precision=_PREC) + bh)
    e = jnp.dot(e, wo, preferred_element_type=jnp.float32, precision=_PREC) + bo
    return jnp.broadcast_to(e, (e.shape[0], 8))


def _readout_angle_fn(mkj, mji, ageo, w_sbf, w0, b0, w1, b1, wh, bh, wo, bo):
    a = jnp.dot(ageo[:, :N_SBF], w_sbf, preferred_element_type=jnp.float32, precision=_PREC) * (mkj + mji)
    a = _swish(jnp.dot(a, w0, preferred_element_type=jnp.float32, precision=_PREC) + b0)
    a = _swish(jnp.dot(a, w1, preferred_element_type=jnp.float32, precision=_PREC) + b1)
    a = _swish(jnp.dot(a, wh, preferred_element_type=jnp.float32, precision=_PREC) + bh)
    a = jnp.dot(a, wo, preferred_element_type=jnp.float32, precision=_PREC) + bo
    return jnp.broadcast_to(a, (a.shape[0], 8))


def _inter_head_fn(m, geom, wji, bji, wkj, bkj, w_rbf):
    x_ji = _swish(jnp.dot(m, wji, preferred_element_type=jnp.float32, precision=_PREC) + bji)
    x_kj = _swish(jnp.dot(m, wkj, preferred_element_type=jnp.float32, precision=_PREC) + bkj)
    x_kj = x_kj * jnp.dot(geom[:, 1:1 + N_RBF], w_rbf, preferred_element_type=jnp.float32, precision=_PREC)
    return x_ji, x_kj


def _bilinear_fn(xg, ageo, w_sbf, w_bil):
    sbf_w = jnp.dot(ageo[:, :N_SBF], w_sbf, preferred_element_type=jnp.float32, precision=_PREC)
    acc = jnp.zeros_like(xg)
    for l in range(N_BIL):
        acc = acc + sbf_w[:, l:l + 1] * jnp.dot(xg, w_bil[l], preferred_element_type=jnp.float32, precision=_PREC)
    return acc


def _inter_tail_fn(m, x_ji, agg, w1, b1, w2, b2, wo, bo):
    out = x_ji + agg
    out = out + _swish(jnp.dot(out, w1, preferred_element_type=jnp.float32, precision=_PREC) + b1)
    out = out + _swish(jnp.dot(out, w2, preferred_element_type=jnp.float32, precision=_PREC) + b2)
    return m + _swish(jnp.dot(out, wo, preferred_element_type=jnp.float32, precision=_PREC) + bo)


def kernel(nxyz, nbr_list, angle_list, ji_idx, kj_idx, params):
    num_atoms = nxyz.shape[0]
    n_edges = nbr_list.shape[0]
    z = nxyz[:, 0].astype(jnp.int32)
    node_geo = jnp.pad(nxyz[:, 1:4], ((0, 0), (0, 13)))

    g_src = node_geo[nbr_list[:, 0]]
    g_dst = node_geo[nbr_list[:, 1]]
    e_geom = _rowblock_call(_edge_geom_fn, [g_src, g_dst], [], [16])[0]

    a_i = node_geo[angle_list[:, 0]]
    a_j = node_geo[angle_list[:, 1]]
    a_k = node_geo[angle_list[:, 2]]
    a_geo = _rowblock_call(_angle_geom_fn, [a_i, a_j, a_k], [], [8])[0]
    xyz = nxyz[:, 1:]
    d_host = jnp.sqrt(((xyz[nbr_list[:, 0]] - xyz[nbr_list[:, 1]]) ** 2).sum(-1)).reshape(-1, 1)
    a_sbf = _host_sbf(xyz, d_host, angle_list, kj_idx)

    emb_node = params['emb_table'][z]
    e_i = _sc_gather(emb_node, nbr_list[:, 0])
    e_j = _sc_gather(emb_node, nbr_list[:, 1])
    m_ji = _rowblock_call(
        _embed_fn, [e_i, e_j, e_geom],
        [params['emb_rbf']['W'], params['emb_dense']['W'], params['emb_dense']['b']],
        [D])[0]

    def read_edge(blk, m):
        return _rowblock_call(
            _readout_edge_fn, [m, e_geom],
            [blk['edge_rbf']['W'], blk['edge_l0']['W'], blk['edge_l0']['b'],
             blk['edge_l1']['W'], blk['edge_l1']['b'], blk['edge_h']['W'],
             blk['edge_h']['b'], blk['edge_o']['W'], blk['edge_o']['b']],
            [8])[0][:, 0:1]

    def read_angle(blk, m):
        mkj = _sc_gather(m, kj_idx)
        mji = _sc_gather(m, ji_idx)
        return _rowblock_call(
            _readout_angle_fn, [mkj, mji, a_sbf],
            [blk['angle_sbf']['W'], blk['angle_l0']['W'], blk['angle_l0']['b'],
             blk['angle_l1']['W'], blk['angle_l1']['b'], blk['angle_h']['W'],
             blk['angle_h']['b'], blk['angle_o']['W'], blk['angle_o']['b']],
            [8])[0][:, 0:1]

    edge_feats = read_edge(params['readouts'][0], m_ji)
    angle_feats = read_angle(params['readouts'][0], m_ji)

    for i in range(N_CONV):
        blk = params['interactions'][i]
        x_ji, x_kj = _rowblock_call(
            _inter_head_fn, [m_ji, e_geom],
            [blk['dense_ji']['W'], blk['dense_ji']['b'], blk['dense_kj']['W'],
             blk['dense_kj']['b'], blk['dense_rbf']['W']],
            [D, D])
        x_kj_g = _sc_gather(x_kj, kj_idx)
        w_bil = jnp.transpose(blk['w_bil'], (1, 0, 2))
        acc = _rowblock_call(
            _bilinear_fn, [x_kj_g, a_sbf],
            [blk['dense_sbf']['W'], w_bil], [D])[0]
        agg = jax.ops.segment_sum(acc, ji_idx, num_segments=n_edges)
        m_ji = _rowblock_call(
            _inter_tail_fn, [m_ji, x_ji, agg],
            [blk['res1']['W'], blk['res1']['b'], blk['res2']['W'],
             blk['res2']['b'], blk['out']['W'], blk['out']['b']],
            [D])[0]
        edge_feats = edge_feats + read_edge(params['readouts'][i + 1], m_ji)
        angle_feats = angle_feats + read_angle(params['readouts'][i + 1], m_ji)

    _USE_SC_SCATTER = False
    if not _USE_SC_SCATTER:
        f_edge3 = edge_feats * e_geom[:, 7:10]
        f_a_ji3 = angle_feats * a_geo[:, 0:3]
        f_a_jk3 = angle_feats * a_geo[:, 3:6]
        seg = jax.ops.segment_sum
        out3 = (seg(f_edge3, nbr_list[:, 0], num_segments=num_atoms)
                - seg(f_edge3, nbr_list[:, 1], num_segments=num_atoms)
                + seg(f_a_ji3, angle_list[:, 1], num_segments=num_atoms)
                - seg(f_a_ji3, angle_list[:, 0], num_segments=num_atoms)
                + seg(f_a_jk3, angle_list[:, 1], num_segments=num_atoms)
                - seg(f_a_jk3, angle_list[:, 0], num_segments=num_atoms))
        return out3
    f_edge = jnp.pad(edge_feats * e_geom[:, 7:10], ((0, 0), (0, 1))).reshape(-1)
    f_a_ji = jnp.pad(angle_feats * a_geo[:, 0:3], ((0, 0), (0, 1))).reshape(-1)
    f_a_jk = jnp.pad(angle_feats * a_geo[:, 3:6], ((0, 0), (0, 1))).reshape(-1)
    parts = _sc_force_scatter(f_edge, f_a_ji, f_a_jk, nbr_list[:, 0],
                              nbr_list[:, 1], angle_list[:, 0], angle_list[:, 1],
                              num_atoms)

    def sum_body(p_ref, o_ref):
        o_ref[...] = jnp.sum(p_ref[...], axis=0)

    chunk = num_atoms // 10
    out4 = pl.pallas_call(
        sum_body,
        grid=(10,),
        in_specs=[pl.BlockSpec((_NW, chunk, 4), lambda i: (0, i, 0))],
        out_specs=pl.BlockSpec((chunk, 4), lambda i: (i, 0)),
        out_shape=jax.ShapeDtypeStruct((num_atoms, 4), jnp.float32),
    )(parts.reshape(_NW, num_atoms, 4))
    return out4[:, :3]


# speedup vs baseline: 1.0343x; 1.0343x over previous
import functools

import jax
import jax.numpy as jnp
import numpy as np
from jax import lax
from jax.experimental import pallas as pl
from jax.experimental.pallas import tpu as pltpu
from jax.experimental.pallas import tpu_sc as plsc

_NW = 32

_PREC = jax.lax.Precision.DEFAULT
EPS = 1e-15
N_RBF = 6
CUTOFF = 5.0
P_ENV = 6
L_SPHER = 7
N_SPHER = 6
D = 128
N_BIL = 8
N_CONV = 2
N_SBF = L_SPHER * N_SPHER

_BLK = 1000


def _swish(x):
    return x * jax.nn.sigmoid(x)


def _rowblock_call(fn, row_args, full_args, out_dims, block=_BLK):
    full_args = [a[None] if a.ndim == 1 else a for a in full_args]
    n = row_args[0].shape[0]
    grid = n // block

    def body(*refs):
        n_in = len(row_args) + len(full_args)
        in_refs, out_refs = refs[:n_in], refs[n_in:]
        vals = [r[...] for r in in_refs]
        outs = fn(*vals)
        if not isinstance(outs, (tuple, list)):
            outs = (outs,)
        for o_ref, o in zip(out_refs, outs):
            o_ref[...] = o

    in_specs = []
    for a in row_args:
        bs = (block,) + a.shape[1:]
        in_specs.append(pl.BlockSpec(bs, lambda i, _nd=a.ndim: (i,) + (0,) * (_nd - 1)))
    for a in full_args:
        in_specs.append(pl.BlockSpec(a.shape, lambda i, _nd=a.ndim: (0,) * _nd))
    out_specs = [pl.BlockSpec((block, dm), lambda i: (i, 0)) for dm in out_dims]
    out_shape = [jax.ShapeDtypeStruct((n, dm), jnp.float32) for dm in out_dims]
    res = pl.pallas_call(
        body,
        grid=(grid,),
        in_specs=in_specs,
        out_specs=out_specs,
        out_shape=out_shape,
    )(*row_args, *full_args)
    return res


def _sc_gather(table, idx):
    t_rows, dp = table.shape
    b = idx.shape[0]
    bpw = b // _NW
    n_full, rem = divmod(bpw, 128)
    nch = n_full + (1 if rem else 0)
    mesh = plsc.VectorSubcoreMesh(core_axis_name="c", subcore_axis_name="s")

    @functools.partial(
        pl.kernel, mesh=mesh,
        out_type=jax.ShapeDtypeStruct((b, dp), jnp.float32),
        scratch_types=[
            pltpu.VMEM((bpw,), jnp.int32),
            pltpu.VMEM((128, dp), jnp.float32),
            pltpu.VMEM((128, dp), jnp.float32),
            pltpu.SemaphoreType.DMA,
            pltpu.SemaphoreType.DMA,
        ],
    )
    def k(table_h, idx_h, out_h, idx_v, buf0, buf1, sem0, sem1):
        wid = lax.axis_index("s") * 2 + lax.axis_index("c")
        base = wid * bpw
        pltpu.sync_copy(idx_h.at[pl.ds(base, bpw)], idx_v)
        bufs = (buf0, buf1)
        sems = (sem0, sem1)

        def chunk_len(j):
            return 128 if j < n_full else rem

        prev = None
        for j in range(nch):
            cl = chunk_len(j)
            cp = pltpu.async_copy(
                table_h.at[idx_v.at[pl.ds(j * 128, cl)]],
                bufs[j % 2].at[pl.ds(0, cl)], sems[j % 2])
            if prev is not None:
                prev.wait()
                pl_ = chunk_len(j - 1)
                pltpu.sync_copy(bufs[(j - 1) % 2].at[pl.ds(0, pl_)],
                                out_h.at[pl.ds(base + (j - 1) * 128, pl_)])
            prev = cp
        prev.wait()
        pl_ = chunk_len(nch - 1)
        pltpu.sync_copy(bufs[(nch - 1) % 2].at[pl.ds(0, pl_)],
                        out_h.at[pl.ds(base + (nch - 1) * 128, pl_)])

    return k(table, idx.astype(jnp.int32))


def _envelope(x):
    p = P_ENV
    return (1.0 - ((p + 1) * (p + 2) / 2.0) * x ** p
            + p * (p + 2) * x ** (p + 1)
            - (p * (p + 1) / 2.0) * x ** (p + 2))


def _edge_geom_fn(gi, gj):
    dx = gi[:, 0:1] - gj[:, 0:1]
    dy = gi[:, 1:2] - gj[:, 1:2]
    dz = gi[:, 2:3] - gj[:, 2:3]
    s = dx * dx + dy * dy + dz * dz
    d = jnp.sqrt(s)
    dis = jnp.sqrt(s + EPS)
    x = d / CUTOFF
    env = _envelope(x)
    c = np.sqrt(2.0 / CUTOFF)
    inv = 1.0 / (d + 1e-9)
    rbf = [c * env * jnp.sin(float(n) * np.pi * x) * inv for n in range(1, N_RBF + 1)]
    adj = [dx / dis, dy / dis, dz / dis]
    zero = jnp.zeros_like(d)
    return jnp.concatenate([d] + rbf + adj + [zero] * 6, axis=1)


def _sph_j(l, x):
    x = jnp.where(jnp.abs(x) < 1e-6, 1e-6, x)
    sx, cx = jnp.sin(x), jnp.cos(x)
    j0 = sx / x
    if l == 0:
        return j0
    j1 = sx / (x * x) - cx / x
    jm, jc = j0, j1
    for ll in range(1, l):
        jn = (2 * ll + 1) / x * jc - jm
        jm, jc = jc, jn
    return jc


def _angle_geom_fn(gi, gj, gk):
    jx = [gi[:, c:c + 1] - gj[:, c:c + 1] for c in range(3)]
    kx = [gk[:, c:c + 1] - gj[:, c:c + 1] for c in range(3)]
    s_ji = jx[0] ** 2 + jx[1] ** 2 + jx[2] ** 2
    s_jk = kx[0] ** 2 + kx[1] ** 2 + kx[2] ** 2
    d_ji = jnp.sqrt(s_ji + EPS)
    d_jk = jnp.sqrt(s_jk + EPS)
    u = [jx[c] / d_ji for c in range(3)]
    v = [kx[c] / d_jk for c in range(3)]
    cos_raw = u[0] * v[0] + u[1] * v[1] + u[2] * v[2]
    aa_ji = [(u[c] * cos_raw - v[c]) / d_ji for c in range(3)]
    aa_jk = [(v[c] * cos_raw - u[c]) / d_jk for c in range(3)]
    zero = jnp.zeros_like(d_ji)
    return jnp.concatenate(aa_ji + aa_jk + [zero] * 2, axis=1)


def _host_sbf(xyz, d, angle_list, kj_idx):
    r_ji = xyz[angle_list[:, 0]] - xyz[angle_list[:, 1]]
    r_jk = xyz[angle_list[:, 2]] - xyz[angle_list[:, 1]]
    tn_ji = ((r_ji ** 2 + EPS).sum(-1)) ** 0.5
    tn_jk = ((r_jk ** 2 + EPS).sum(-1)) ** 0.5
    cos_a = (r_ji * r_jk).sum(-1) / (tn_ji * tn_jk)
    alpha = jnp.arccos(jnp.clip(cos_a, -1.0 + 1e-7, 1.0 - 1e-7))
    x = (d[kj_idx] / CUTOFF)[:, 0]
    env = _envelope(x)
    cos_al = jnp.cos(alpha)
    P = [jnp.ones_like(cos_al), cos_al]
    for l in range(1, L_SPHER - 1):
        P.append(((2 * l + 1) * cos_al * P[l] - l * P[l - 1]) / (l + 1))
    feats = []
    for l in range(L_SPHER):
        for n in range(1, N_SPHER + 1):
            z = np.pi * (n + l / 2.0)
            feats.append(env * _sph_j(l, z * x) * P[l])
    return jnp.stack(feats, axis=-1)


def _embed_fn(ei, ej, geom, w_rbf, w_emb, b_emb):
    e_d = jnp.dot(geom[:, 1:1 + N_RBF], w_rbf, preferred_element_type=jnp.float32, precision=_PREC)
    cat = jnp.concatenate([ei, ej, e_d], axis=1)
    return _swish(jnp.dot(cat, w_emb, preferred_element_type=jnp.float32, precision=_PREC) + b_emb)


def _readout_edge_fn(m, geom, w_rbf, w0, b0, w1, b1, wh, bh, wo, bo):
    e = jnp.dot(geom[:, 1:1 + N_RBF], w_rbf, preferred_element_type=jnp.float32, precision=_PREC) * m
    e = _swish(jnp.dot(e, w0, preferred_element_type=jnp.float32, precision=_PREC) + b0)
    e = _swish(jnp.dot(e, w1, preferred_element_type=jnp.float32, precision=_PREC) + b1)
    e = _swish(jnp.dot(e, wh, preferred_element_type=jnp.float32, precision=_PREC) + bh)
    e = jnp.dot(e, wo, preferred_element_type=jnp.float32, precision=_PREC) + bo
    return jnp.broadcast_to(e, (e.shape[0], 8))


def _readout_angle_fn(mkj, mji, ageo, w_sbf, w0, b0, w1, b1, wh, bh, wo, bo):
    a = jnp.dot(ageo[:, :N_SBF], w_sbf, preferred_element_type=jnp.float32, precision=_PREC) * (mkj + mji)
    a = _swish(jnp.dot(a, w0, preferred_element_type=jnp.float32, precision=_PREC) + b0)
    a = _swish(jnp.dot(a, w1, preferred_element_type=jnp.float32, precision=_PREC) + b1)
    a = _swish(jnp.dot(a, wh, preferred_element_type=jnp.float32, precision=_PREC) + bh)
    a = jnp.dot(a, wo, preferred_element_type=jnp.float32, precision=_PREC) + bo
    return jnp.broadcast_to(a, (a.shape[0], 8))


def _inter_head_fn(m, geom, wji, bji, wkj, bkj, w_rbf):
    x_ji = _swish(jnp.dot(m, wji, preferred_element_type=jnp.float32, precision=_PREC) + bji)
    x_kj = _swish(jnp.dot(m, wkj, preferred_element_type=jnp.float32, precision=_PREC) + bkj)
    x_kj = x_kj * jnp.dot(geom[:, 1:1 + N_RBF], w_rbf, preferred_element_type=jnp.float32, precision=_PREC)
    return x_ji, x_kj


def _bilinear_fn(xg, ageo, w_sbf, w_bil):
    sbf_w = jnp.dot(ageo[:, :N_SBF], w_sbf, preferred_element_type=jnp.float32, precision=_PREC)
    acc = jnp.zeros_like(xg)
    for l in range(N_BIL):
        acc = acc + sbf_w[:, l:l + 1] * jnp.dot(xg, w_bil[l], preferred_element_type=jnp.float32, precision=_PREC)
    return acc


def _inter_tail_fn(m, x_ji, agg, w1, b1, w2, b2, wo, bo):
    out = x_ji + agg
    out = out + _swish(jnp.dot(out, w1, preferred_element_type=jnp.float32, precision=_PREC) + b1)
    out = out + _swish(jnp.dot(out, w2, preferred_element_type=jnp.float32, precision=_PREC) + b2)
    return m + _swish(jnp.dot(out, wo, preferred_element_type=jnp.float32, precision=_PREC) + bo)


def _ro_edge(m, geom, w_rbf, w0, b0, w1, b1, wh, bh, wo, bo):
    e = jnp.dot(geom[:, 1:1 + N_RBF], w_rbf, preferred_element_type=jnp.float32, precision=_PREC) * m
    e = _swish(jnp.dot(e, w0, preferred_element_type=jnp.float32, precision=_PREC) + b0)
    e = _swish(jnp.dot(e, w1, preferred_element_type=jnp.float32, precision=_PREC) + b1)
    e = _swish(jnp.dot(e, wh, preferred_element_type=jnp.float32, precision=_PREC) + bh)
    e = jnp.dot(e, wo, preferred_element_type=jnp.float32, precision=_PREC) + bo
    return jnp.broadcast_to(e, (e.shape[0], 8))


def _embed_head_fn(ei, ej, geom, w_rbf_e, w_emb, b_emb,
                   r_rbf, r0, rb0, r1, rb1, rh, rbh, ro, rbo,
                   wji, bji, wkj, bkj, w_rbf_i):
    e_d = jnp.dot(geom[:, 1:1 + N_RBF], w_rbf_e, preferred_element_type=jnp.float32, precision=_PREC)
    cat = jnp.concatenate([ei, ej, e_d], axis=1)
    m = _swish(jnp.dot(cat, w_emb, preferred_element_type=jnp.float32, precision=_PREC) + b_emb)
    ef = _ro_edge(m, geom, r_rbf, r0, rb0, r1, rb1, rh, rbh, ro, rbo)
    x_ji = _swish(jnp.dot(m, wji, preferred_element_type=jnp.float32, precision=_PREC) + bji)
    x_kj = _swish(jnp.dot(m, wkj, preferred_element_type=jnp.float32, precision=_PREC) + bkj)
    x_kj = x_kj * jnp.dot(geom[:, 1:1 + N_RBF], w_rbf_i, preferred_element_type=jnp.float32, precision=_PREC)
    return m, ef, x_ji, x_kj


def _tail_ro_head_fn(m, x_ji, agg, geom, w1, b1, w2, b2, wo, bo,
                     r_rbf, r0, rb0, r1, rb1, rh, rbh, ro, rbo,
                     wji, bji, wkj, bkj, w_rbf_i):
    m_new = _inter_tail_fn(m, x_ji, agg, w1, b1, w2, b2, wo, bo)
    ef = _ro_edge(m_new, geom, r_rbf, r0, rb0, r1, rb1, rh, rbh, ro, rbo)
    x_ji2 = _swish(jnp.dot(m_new, wji, preferred_element_type=jnp.float32, precision=_PREC) + bji)
    x_kj2 = _swish(jnp.dot(m_new, wkj, preferred_element_type=jnp.float32, precision=_PREC) + bkj)
    x_kj2 = x_kj2 * jnp.dot(geom[:, 1:1 + N_RBF], w_rbf_i, preferred_element_type=jnp.float32, precision=_PREC)
    return m_new, ef, x_ji2, x_kj2


def _tail_ro_fn(m, x_ji, agg, geom, w1, b1, w2, b2, wo, bo,
                r_rbf, r0, rb0, r1, rb1, rh, rbh, ro, rbo):
    m_new = _inter_tail_fn(m, x_ji, agg, w1, b1, w2, b2, wo, bo)
    ef = _ro_edge(m_new, geom, r_rbf, r0, rb0, r1, rb1, rh, rbh, ro, rbo)
    return m_new, ef


def kernel(nxyz, nbr_list, angle_list, ji_idx, kj_idx, params):
    num_atoms = nxyz.shape[0]
    n_edges = nbr_list.shape[0]
    z = nxyz[:, 0].astype(jnp.int32)
    node_geo = jnp.pad(nxyz[:, 1:4], ((0, 0), (0, 13)))

    g_src = node_geo[nbr_list[:, 0]]
    g_dst = node_geo[nbr_list[:, 1]]
    e_geom = _rowblock_call(_edge_geom_fn, [g_src, g_dst], [], [16])[0]

    a_i = node_geo[angle_list[:, 0]]
    a_j = node_geo[angle_list[:, 1]]
    a_k = node_geo[angle_list[:, 2]]
    a_geo = _rowblock_call(_angle_geom_fn, [a_i, a_j, a_k], [], [8])[0]
    xyz = nxyz[:, 1:]
    d_host = jnp.sqrt(((xyz[nbr_list[:, 0]] - xyz[nbr_list[:, 1]]) ** 2).sum(-1)).reshape(-1, 1)
    a_sbf = _host_sbf(xyz, d_host, angle_list, kj_idx)

    emb_node = params['emb_table'][z]
    e_i = _sc_gather(emb_node, nbr_list[:, 0])
    e_j = _sc_gather(emb_node, nbr_list[:, 1])

    def ro_w(blk):
        return [blk['edge_rbf']['W'], blk['edge_l0']['W'], blk['edge_l0']['b'],
                blk['edge_l1']['W'], blk['edge_l1']['b'], blk['edge_h']['W'],
                blk['edge_h']['b'], blk['edge_o']['W'], blk['edge_o']['b']]

    def head_w(blk):
        return [blk['dense_ji']['W'], blk['dense_ji']['b'], blk['dense_kj']['W'],
                blk['dense_kj']['b'], blk['dense_rbf']['W']]

    def tail_w(blk):
        return [blk['res1']['W'], blk['res1']['b'], blk['res2']['W'],
                blk['res2']['b'], blk['out']['W'], blk['out']['b']]

    def read_angle(blk, m):
        mkj = _sc_gather(m, kj_idx)
        mji = _sc_gather(m, ji_idx)
        return _rowblock_call(
            _readout_angle_fn, [mkj, mji, a_sbf],
            [blk['angle_sbf']['W'], blk['angle_l0']['W'], blk['angle_l0']['b'],
             blk['angle_l1']['W'], blk['angle_l1']['b'], blk['angle_h']['W'],
             blk['angle_h']['b'], blk['angle_o']['W'], blk['angle_o']['b']],
            [8])[0][:, 0:1]

    def agg_of(blk, x_kj):
        x_kj_g = _sc_gather(x_kj, kj_idx)
        w_bil = jnp.transpose(blk['w_bil'], (1, 0, 2))
        acc = _rowblock_call(
            _bilinear_fn, [x_kj_g, a_sbf],
            [blk['dense_sbf']['W'], w_bil], [D])[0]
        return jax.ops.segment_sum(acc, ji_idx, num_segments=n_edges)

    ro, ib = params['readouts'], params['interactions']
    m0, ef0, x_ji, x_kj = _rowblock_call(
        _embed_head_fn, [e_i, e_j, e_geom],
        [params['emb_rbf']['W'], params['emb_dense']['W'], params['emb_dense']['b']]
        + ro_w(ro[0]) + head_w(ib[0]),
        [D, 8, D, D])
    edge_feats = ef0[:, 0:1]
    angle_feats = read_angle(ro[0], m0)

    agg = agg_of(ib[0], x_kj)
    m1, ef1, x_ji, x_kj = _rowblock_call(
        _tail_ro_head_fn, [m0, x_ji, agg, e_geom],
        tail_w(ib[0]) + ro_w(ro[1]) + head_w(ib[1]),
        [D, 8, D, D])
    edge_feats = edge_feats + ef1[:, 0:1]
    angle_feats = angle_feats + read_angle(ro[1], m1)

    agg = agg_of(ib[1], x_kj)
    m2, ef2 = _rowblock_call(
        _tail_ro_fn, [m1, x_ji, agg, e_geom],
        tail_w(ib[1]) + ro_w(ro[2]),
        [D, 8])
    edge_feats = edge_feats + ef2[:, 0:1]
    angle_feats = angle_feats + read_angle(ro[2], m2)

    f_edge3 = edge_feats * e_geom[:, 7:10]
    f_a_ji3 = angle_feats * a_geo[:, 0:3]
    f_a_jk3 = angle_feats * a_geo[:, 3:6]
    seg = jax.ops.segment_sum
    out3 = (seg(f_edge3, nbr_list[:, 0], num_segments=num_atoms)
            - seg(f_edge3, nbr_list[:, 1], num_segments=num_atoms)
            + seg(f_a_ji3, angle_list[:, 1], num_segments=num_atoms)
            - seg(f_a_ji3, angle_list[:, 0], num_segments=num_atoms)
            + seg(f_a_jk3, angle_list[:, 1], num_segments=num_atoms)
            - seg(f_a_jk3, angle_list[:, 0], num_segments=num_atoms))
    return out3


# speedup vs baseline: 1.0779x; 1.0422x over previous
import functools

import jax
import jax.numpy as jnp
import numpy as np
from jax import lax
from jax.experimental import pallas as pl
from jax.experimental.pallas import tpu as pltpu
from jax.experimental.pallas import tpu_sc as plsc

_NW = 32

_PREC = jax.lax.Precision.DEFAULT
EPS = 1e-15
N_RBF = 6
CUTOFF = 5.0
P_ENV = 6
L_SPHER = 7
N_SPHER = 6
D = 128
N_BIL = 8
N_CONV = 2
N_SBF = L_SPHER * N_SPHER

_BLK = 2000


def _swish(x):
    return x * jax.nn.sigmoid(x)


def _rowblock_call(fn, row_args, full_args, out_dims, block=_BLK):
    full_args = [a[None] if a.ndim == 1 else a for a in full_args]
    n = row_args[0].shape[0]
    grid = n // block

    def body(*refs):
        n_in = len(row_args) + len(full_args)
        in_refs, out_refs = refs[:n_in], refs[n_in:]
        vals = [r[...] for r in in_refs]
        outs = fn(*vals)
        if not isinstance(outs, (tuple, list)):
            outs = (outs,)
        for o_ref, o in zip(out_refs, outs):
            o_ref[...] = o

    in_specs = []
    for a in row_args:
        bs = (block,) + a.shape[1:]
        in_specs.append(pl.BlockSpec(bs, lambda i, _nd=a.ndim: (i,) + (0,) * (_nd - 1)))
    for a in full_args:
        in_specs.append(pl.BlockSpec(a.shape, lambda i, _nd=a.ndim: (0,) * _nd))
    out_specs = [pl.BlockSpec((block, dm), lambda i: (i, 0)) for dm in out_dims]
    out_shape = [jax.ShapeDtypeStruct((n, dm), jnp.float32) for dm in out_dims]
    res = pl.pallas_call(
        body,
        grid=(grid,),
        in_specs=in_specs,
        out_specs=out_specs,
        out_shape=out_shape,
    )(*row_args, *full_args)
    return res


def _sc_gather(table, idx):
    t_rows, dp = table.shape
    b = idx.shape[0]
    bpw = b // _NW
    n_full, rem = divmod(bpw, 128)
    nch = n_full + (1 if rem else 0)
    mesh = plsc.VectorSubcoreMesh(core_axis_name="c", subcore_axis_name="s")

    @functools.partial(
        pl.kernel, mesh=mesh,
        out_type=jax.ShapeDtypeStruct((b, dp), jnp.float32),
        scratch_types=[
            pltpu.VMEM((bpw,), jnp.int32),
            pltpu.VMEM((128, dp), jnp.float32),
            pltpu.VMEM((128, dp), jnp.float32),
            pltpu.SemaphoreType.DMA,
            pltpu.SemaphoreType.DMA,
        ],
    )
    def k(table_h, idx_h, out_h, idx_v, buf0, buf1, sem0, sem1):
        wid = lax.axis_index("s") * 2 + lax.axis_index("c")
        base = wid * bpw
        pltpu.sync_copy(idx_h.at[pl.ds(base, bpw)], idx_v)
        bufs = (buf0, buf1)
        sems = (sem0, sem1)

        def chunk_len(j):
            return 128 if j < n_full else rem

        prev = None
        for j in range(nch):
            cl = chunk_len(j)
            cp = pltpu.async_copy(
                table_h.at[idx_v.at[pl.ds(j * 128, cl)]],
                bufs[j % 2].at[pl.ds(0, cl)], sems[j % 2])
            if prev is not None:
                prev.wait()
                pl_ = chunk_len(j - 1)
                pltpu.sync_copy(bufs[(j - 1) % 2].at[pl.ds(0, pl_)],
                                out_h.at[pl.ds(base + (j - 1) * 128, pl_)])
            prev = cp
        prev.wait()
        pl_ = chunk_len(nch - 1)
        pltpu.sync_copy(bufs[(nch - 1) % 2].at[pl.ds(0, pl_)],
                        out_h.at[pl.ds(base + (nch - 1) * 128, pl_)])

    return k(table, idx.astype(jnp.int32))


def _envelope(x):
    p = P_ENV
    return (1.0 - ((p + 1) * (p + 2) / 2.0) * x ** p
            + p * (p + 2) * x ** (p + 1)
            - (p * (p + 1) / 2.0) * x ** (p + 2))


def _edge_geom_fn(gi, gj):
    dx = gi[:, 0:1] - gj[:, 0:1]
    dy = gi[:, 1:2] - gj[:, 1:2]
    dz = gi[:, 2:3] - gj[:, 2:3]
    s = dx * dx + dy * dy + dz * dz
    d = jnp.sqrt(s)
    dis = jnp.sqrt(s + EPS)
    x = d / CUTOFF
    env = _envelope(x)
    c = np.sqrt(2.0 / CUTOFF)
    inv = 1.0 / (d + 1e-9)
    rbf = [c * env * jnp.sin(float(n) * np.pi * x) * inv for n in range(1, N_RBF + 1)]
    adj = [dx / dis, dy / dis, dz / dis]
    zero = jnp.zeros_like(d)
    return jnp.concatenate([d] + rbf + adj + [zero] * 6, axis=1)


def _sph_j(l, x):
    x = jnp.where(jnp.abs(x) < 1e-6, 1e-6, x)
    sx, cx = jnp.sin(x), jnp.cos(x)
    j0 = sx / x
    if l == 0:
        return j0
    j1 = sx / (x * x) - cx / x
    jm, jc = j0, j1
    for ll in range(1, l):
        jn = (2 * ll + 1) / x * jc - jm
        jm, jc = jc, jn
    return jc


def _angle_geom_fn(gi, gj, gk):
    jx = [gi[:, c:c + 1] - gj[:, c:c + 1] for c in range(3)]
    kx = [gk[:, c:c + 1] - gj[:, c:c + 1] for c in range(3)]
    s_ji = jx[0] ** 2 + jx[1] ** 2 + jx[2] ** 2
    s_jk = kx[0] ** 2 + kx[1] ** 2 + kx[2] ** 2
    d_ji = jnp.sqrt(s_ji + EPS)
    d_jk = jnp.sqrt(s_jk + EPS)
    u = [jx[c] / d_ji for c in range(3)]
    v = [kx[c] / d_jk for c in range(3)]
    cos_raw = u[0] * v[0] + u[1] * v[1] + u[2] * v[2]
    aa_ji = [(u[c] * cos_raw - v[c]) / d_ji for c in range(3)]
    aa_jk = [(v[c] * cos_raw - u[c]) / d_jk for c in range(3)]
    zero = jnp.zeros_like(d_ji)
    return jnp.concatenate(aa_ji + aa_jk + [zero] * 2, axis=1)


def _host_sbf(xyz, d, angle_list, kj_idx):
    r_ji = xyz[angle_list[:, 0]] - xyz[angle_list[:, 1]]
    r_jk = xyz[angle_list[:, 2]] - xyz[angle_list[:, 1]]
    tn_ji = ((r_ji ** 2 + EPS).sum(-1)) ** 0.5
    tn_jk = ((r_jk ** 2 + EPS).sum(-1)) ** 0.5
    cos_a = (r_ji * r_jk).sum(-1) / (tn_ji * tn_jk)
    alpha = jnp.arccos(jnp.clip(cos_a, -1.0 + 1e-7, 1.0 - 1e-7))
    x = (d[kj_idx] / CUTOFF)[:, 0]
    env = _envelope(x)
    cos_al = jnp.cos(alpha)
    P = [jnp.ones_like(cos_al), cos_al]
    for l in range(1, L_SPHER - 1):
        P.append(((2 * l + 1) * cos_al * P[l] - l * P[l - 1]) / (l + 1))
    feats = []
    for l in range(L_SPHER):
        for n in range(1, N_SPHER + 1):
            z = np.pi * (n + l / 2.0)
            feats.append(env * _sph_j(l, z * x) * P[l])
    return jnp.stack(feats, axis=-1)


def _embed_fn(ei, ej, geom, w_rbf, w_emb, b_emb):
    e_d = jnp.dot(geom[:, 1:1 + N_RBF], w_rbf, preferred_element_type=jnp.float32, precision=_PREC)
    cat = jnp.concatenate([ei, ej, e_d], axis=1)
    return _swish(jnp.dot(cat, w_emb, preferred_element_type=jnp.float32, precision=_PREC) + b_emb)


def _readout_edge_fn(m, geom, w_rbf, w0, b0, w1, b1, wh, bh, wo, bo):
    e = jnp.dot(geom[:, 1:1 + N_RBF], w_rbf, preferred_element_type=jnp.float32, precision=_PREC) * m
    e = _swish(jnp.dot(e, w0, preferred_element_type=jnp.float32, precision=_PREC) + b0)
    e = _swish(jnp.dot(e, w1, preferred_element_type=jnp.float32, precision=_PREC) + b1)
    e = _swish(jnp.dot(e, wh, preferred_element_type=jnp.float32, precision=_PREC) + bh)
    e = jnp.dot(e, wo, preferred_element_type=jnp.float32, precision=_PREC) + bo
    return jnp.broadcast_to(e, (e.shape[0], 8))


def _readout_angle_fn(mkj, mji, ageo, w_sbf, w0, b0, w1, b1, wh, bh, wo, bo):
    a = jnp.dot(ageo[:, :N_SBF], w_sbf, preferred_element_type=jnp.float32, precision=_PREC) * (mkj + mji)
    a = _swish(jnp.dot(a, w0, preferred_element_type=jnp.float32, precision=_PREC) + b0)
    a = _swish(jnp.dot(a, w1, preferred_element_type=jnp.float32, precision=_PREC) + b1)
    a = _swish(jnp.dot(a, wh, preferred_element_type=jnp.float32, precision=_PREC) + bh)
    a = jnp.dot(a, wo, preferred_element_type=jnp.float32, precision=_PREC) + bo
    return jnp.broadcast_to(a, (a.shape[0], 8))


def _inter_head_fn(m, geom, wji, bji, wkj, bkj, w_rbf):
    x_ji = _swish(jnp.dot(m, wji, preferred_element_type=jnp.float32, precision=_PREC) + bji)
    x_kj = _swish(jnp.dot(m, wkj, preferred_element_type=jnp.float32, precision=_PREC) + bkj)
    x_kj = x_kj * jnp.dot(geom[:, 1:1 + N_RBF], w_rbf, preferred_element_type=jnp.float32, precision=_PREC)
    return x_ji, x_kj


def _bilinear_fn(xg, ageo, w_sbf, w_bil):
    sbf_w = jnp.dot(ageo[:, :N_SBF], w_sbf, preferred_element_type=jnp.float32, precision=_PREC)
    acc = jnp.zeros_like(xg)
    for l in range(N_BIL):
        acc = acc + sbf_w[:, l:l + 1] * jnp.dot(xg, w_bil[l], preferred_element_type=jnp.float32, precision=_PREC)
    return acc


def _inter_tail_fn(m, x_ji, agg, w1, b1, w2, b2, wo, bo):
    out = x_ji + agg
    out = out + _swish(jnp.dot(out, w1, preferred_element_type=jnp.float32, precision=_PREC) + b1)
    out = out + _swish(jnp.dot(out, w2, preferred_element_type=jnp.float32, precision=_PREC) + b2)
    return m + _swish(jnp.dot(out, wo, preferred_element_type=jnp.float32, precision=_PREC) + bo)


def _ro_edge(m, geom, w_rbf, w0, b0, w1, b1, wh, bh, wo, bo):
    e = jnp.dot(geom[:, 1:1 + N_RBF], w_rbf, preferred_element_type=jnp.float32, precision=_PREC) * m
    e = _swish(jnp.dot(e, w0, preferred_element_type=jnp.float32, precision=_PREC) + b0)
    e = _swish(jnp.dot(e, w1, preferred_element_type=jnp.float32, precision=_PREC) + b1)
    e = _swish(jnp.dot(e, wh, preferred_element_type=jnp.float32, precision=_PREC) + bh)
    e = jnp.dot(e, wo, preferred_element_type=jnp.float32, precision=_PREC) + bo
    return jnp.broadcast_to(e, (e.shape[0], 8))


def _embed_head_fn(ei, ej, geom, w_rbf_e, w_emb, b_emb,
                   r_rbf, r0, rb0, r1, rb1, rh, rbh, ro, rbo,
                   wji, bji, wkj, bkj, w_rbf_i):
    e_d = jnp.dot(geom[:, 1:1 + N_RBF], w_rbf_e, preferred_element_type=jnp.float32, precision=_PREC)
    cat = jnp.concatenate([ei, ej, e_d], axis=1)
    m = _swish(jnp.dot(cat, w_emb, preferred_element_type=jnp.float32, precision=_PREC) + b_emb)
    ef = _ro_edge(m, geom, r_rbf, r0, rb0, r1, rb1, rh, rbh, ro, rbo)
    x_ji = _swish(jnp.dot(m, wji, preferred_element_type=jnp.float32, precision=_PREC) + bji)
    x_kj = _swish(jnp.dot(m, wkj, preferred_element_type=jnp.float32, precision=_PREC) + bkj)
    x_kj = x_kj * jnp.dot(geom[:, 1:1 + N_RBF], w_rbf_i, preferred_element_type=jnp.float32, precision=_PREC)
    return m, ef, x_ji, x_kj


def _tail_ro_head_fn(m, x_ji, agg, geom, w1, b1, w2, b2, wo, bo,
                     r_rbf, r0, rb0, r1, rb1, rh, rbh, ro, rbo,
                     wji, bji, wkj, bkj, w_rbf_i):
    m_new = _inter_tail_fn(m, x_ji, agg, w1, b1, w2, b2, wo, bo)
    ef = _ro_edge(m_new, geom, r_rbf, r0, rb0, r1, rb1, rh, rbh, ro, rbo)
    x_ji2 = _swish(jnp.dot(m_new, wji, preferred_element_type=jnp.float32, precision=_PREC) + bji)
    x_kj2 = _swish(jnp.dot(m_new, wkj, preferred_element_type=jnp.float32, precision=_PREC) + bkj)
    x_kj2 = x_kj2 * jnp.dot(geom[:, 1:1 + N_RBF], w_rbf_i, preferred_element_type=jnp.float32, precision=_PREC)
    return m_new, ef, x_ji2, x_kj2


def _tail_ro_fn(m, x_ji, agg, geom, w1, b1, w2, b2, wo, bo,
                r_rbf, r0, rb0, r1, rb1, rh, rbh, ro, rbo):
    m_new = _inter_tail_fn(m, x_ji, agg, w1, b1, w2, b2, wo, bo)
    ef = _ro_edge(m_new, geom, r_rbf, r0, rb0, r1, rb1, rh, rbh, ro, rbo)
    return m_new, ef


def kernel(nxyz, nbr_list, angle_list, ji_idx, kj_idx, params):
    num_atoms = nxyz.shape[0]
    n_edges = nbr_list.shape[0]
    z = nxyz[:, 0].astype(jnp.int32)
    node_geo = jnp.pad(nxyz[:, 1:4], ((0, 0), (0, 13)))

    g_src = node_geo[nbr_list[:, 0]]
    g_dst = node_geo[nbr_list[:, 1]]
    e_geom = _rowblock_call(_edge_geom_fn, [g_src, g_dst], [], [16])[0]

    a_i = node_geo[angle_list[:, 0]]
    a_j = node_geo[angle_list[:, 1]]
    a_k = node_geo[angle_list[:, 2]]
    a_geo = _rowblock_call(_angle_geom_fn, [a_i, a_j, a_k], [], [8])[0]
    xyz = nxyz[:, 1:]
    d_host = jnp.sqrt(((xyz[nbr_list[:, 0]] - xyz[nbr_list[:, 1]]) ** 2).sum(-1)).reshape(-1, 1)
    a_sbf = _host_sbf(xyz, d_host, angle_list, kj_idx)

    emb_node = params['emb_table'][z]
    e_i = _sc_gather(emb_node, nbr_list[:, 0])
    e_j = _sc_gather(emb_node, nbr_list[:, 1])

    def ro_w(blk):
        return [blk['edge_rbf']['W'], blk['edge_l0']['W'], blk['edge_l0']['b'],
                blk['edge_l1']['W'], blk['edge_l1']['b'], blk['edge_h']['W'],
                blk['edge_h']['b'], blk['edge_o']['W'], blk['edge_o']['b']]

    def head_w(blk):
        return [blk['dense_ji']['W'], blk['dense_ji']['b'], blk['dense_kj']['W'],
                blk['dense_kj']['b'], blk['dense_rbf']['W']]

    def tail_w(blk):
        return [blk['res1']['W'], blk['res1']['b'], blk['res2']['W'],
                blk['res2']['b'], blk['out']['W'], blk['out']['b']]

    def read_angle(blk, m):
        mkj = _sc_gather(m, kj_idx)
        mji = _sc_gather(m, ji_idx)
        return _rowblock_call(
            _readout_angle_fn, [mkj, mji, a_sbf],
            [blk['angle_sbf']['W'], blk['angle_l0']['W'], blk['angle_l0']['b'],
             blk['angle_l1']['W'], blk['angle_l1']['b'], blk['angle_h']['W'],
             blk['angle_h']['b'], blk['angle_o']['W'], blk['angle_o']['b']],
            [8])[0][:, 0:1]

    def agg_of(blk, x_kj):
        x_kj_g = _sc_gather(x_kj, kj_idx)
        w_bil = jnp.transpose(blk['w_bil'], (1, 0, 2))
        acc = _rowblock_call(
            _bilinear_fn, [x_kj_g, a_sbf],
            [blk['dense_sbf']['W'], w_bil], [D])[0]
        return jax.ops.segment_sum(acc, ji_idx, num_segments=n_edges)

    ro, ib = params['readouts'], params['interactions']
    m0, ef0, x_ji, x_kj = _rowblock_call(
        _embed_head_fn, [e_i, e_j, e_geom],
        [params['emb_rbf']['W'], params['emb_dense']['W'], params['emb_dense']['b']]
        + ro_w(ro[0]) + head_w(ib[0]),
        [D, 8, D, D])
    edge_feats = ef0[:, 0:1]
    angle_feats = read_angle(ro[0], m0)

    agg = agg_of(ib[0], x_kj)
    m1, ef1, x_ji, x_kj = _rowblock_call(
        _tail_ro_head_fn, [m0, x_ji, agg, e_geom],
        tail_w(ib[0]) + ro_w(ro[1]) + head_w(ib[1]),
        [D, 8, D, D])
    edge_feats = edge_feats + ef1[:, 0:1]
    angle_feats = angle_feats + read_angle(ro[1], m1)

    agg = agg_of(ib[1], x_kj)
    m2, ef2 = _rowblock_call(
        _tail_ro_fn, [m1, x_ji, agg, e_geom],
        tail_w(ib[1]) + ro_w(ro[2]),
        [D, 8])
    edge_feats = edge_feats + ef2[:, 0:1]
    angle_feats = angle_feats + read_angle(ro[2], m2)

    f_edge3 = edge_feats * e_geom[:, 7:10]
    f_a_ji3 = angle_feats * a_geo[:, 0:3]
    f_a_jk3 = angle_feats * a_geo[:, 3:6]
    seg = jax.ops.segment_sum
    out3 = (seg(f_edge3, nbr_list[:, 0], num_segments=num_atoms)
            - seg(f_edge3, nbr_list[:, 1], num_segments=num_atoms)
            + seg(f_a_ji3, angle_list[:, 1], num_segments=num_atoms)
            - seg(f_a_ji3, angle_list[:, 0], num_segments=num_atoms)
            + seg(f_a_jk3, angle_list[:, 1], num_segments=num_atoms)
            - seg(f_a_jk3, angle_list[:, 0], num_segments=num_atoms))
    return out3
